# baseline (speedup 1.0000x reference)
import functools

import jax
import jax.numpy as jnp
from jax import lax
from jax.experimental import pallas as pl
from jax.experimental.pallas import tpu as pltpu
from jax.experimental.pallas import tpu_sc as plsc

CLS = 100000
FEAT = 64
B = 16384
NBLK = CLS // 8

NCORE = 2
NSUB = 16
NW = NCORE * NSUB
CHUNK = B // NW
BSZ = 32
NBATCH = CHUNK // BSZ
CLS_PAD = 100352
STRIPE = CLS_PAD // NSUB


def _main_body(xs_hbm, ys_hbm, center_hbm, cnt_hbm, dsum_hbm,
               cnt_sh, dsum_sh,
               idx_v, xs_v, tiles, dvals, ones_v, stage, sem, semx):
    cid = lax.axis_index("c")
    sid = lax.axis_index("s")
    wid = cid * NSUB + sid
    zero16 = jnp.zeros((16,), jnp.float32)
    one16 = jnp.ones((16,), jnp.float32)
    lane = lax.iota(jnp.int32, 16)

    def fire(b):
        ring = (b & 1) * BSZ
        pltpu.async_copy(
            xs_hbm.at[pl.ds((wid * CHUNK + b * BSZ) * FEAT, BSZ * FEAT)],
            xs_v.at[pl.ds((b & 1) * BSZ * FEAT, BSZ * FEAT)], semx)
        for k in range(BSZ // 16):
            yv = idx_v[b, pl.ds(k * 16, 16)]
            for l in range(16):
                y = yv[l]
                pltpu.async_copy(center_hbm.at[pl.ds((y >> 3) * 8, 8)],
                                 tiles.at[ring + k * 16 + l], sem)

    def drain(b):
        pltpu.make_async_copy(
            xs_hbm.at[pl.ds(0, BSZ * FEAT)],
            xs_v.at[pl.ds((b & 1) * BSZ * FEAT, BSZ * FEAT)], semx).wait()
        for n in range(BSZ):
            pltpu.make_async_copy(
                center_hbm.at[pl.ds(0, 8)],
                tiles.at[(b & 1) * BSZ + n], sem).wait()

    for b in range(NBATCH):
        pltpu.sync_copy(ys_hbm.at[pl.ds(wid * CHUNK + b * BSZ, BSZ)],
                        idx_v.at[b])

    def zfill(i, _):
        stage[pl.ds(i * 16, 16)] = zero16
        return 0
    lax.fori_loop(0, STRIPE // 16, zfill, 0)
    pltpu.sync_copy(stage.at[pl.ds(0, STRIPE)],
                    cnt_sh.at[pl.ds(sid * STRIPE, STRIPE)])
    pltpu.sync_copy(stage.at[pl.ds(0, STRIPE)],
                    dsum_sh.at[pl.ds(sid * STRIPE, STRIPE)])
    for k in range(BSZ // 16):
        ones_v[0, pl.ds(k * 16, 16)] = one16

    fire(0)
    plsc.subcore_barrier()

    def batch_step(b, _):
        @pl.when(b + 1 < NBATCH)
        def _():
            fire(b + 1)
        drain(b)
        ring = (b & 1) * BSZ

        def dgroup(k, _):
            yv = idx_v[b, pl.ds(k * 16, 16)]
            slot = ring + k * 16 + lane
            row = yv & 7
            xbase = ((b & 1) * BSZ + k * 16 + lane) * FEAT
            acc = jnp.zeros((16,), jnp.float32)
            for f0 in range(FEAT):
                fd = (lane + f0) & (FEAT - 1)
                xv = plsc.load_gather(xs_v, [xbase + fd])
                cv = plsc.load_gather(tiles, [slot, row, fd])
                df = xv - cv
                acc = acc + df * df
            dvals[0, pl.ds(k * 16, 16)] = acc
            return 0
        lax.fori_loop(0, BSZ // 16, dgroup, 0)

        pltpu.sync_copy(ones_v.at[0], cnt_sh.at[idx_v.at[b]], add=True)
        pltpu.sync_copy(dvals.at[0], dsum_sh.at[idx_v.at[b]], add=True)
        return 0
    lax.fori_loop(0, NBATCH, batch_step, 0)

    plsc.subcore_barrier()

    pltpu.sync_copy(cnt_sh.at[pl.ds(sid * STRIPE, STRIPE)], stage)
    pltpu.sync_copy(stage,
                    cnt_hbm.at[pl.ds(cid * CLS_PAD + sid * STRIPE, STRIPE)])
    pltpu.sync_copy(dsum_sh.at[pl.ds(sid * STRIPE, STRIPE)], stage)
    pltpu.sync_copy(stage,
                    dsum_hbm.at[pl.ds(cid * CLS_PAD + sid * STRIPE, STRIPE)])


def _reduce_body(cnt_hbm, dsum_hbm, out_hbm,
                 part_sh, n0, n1, d0, d1, fin_v):
    cid = lax.axis_index("c")
    sid = lax.axis_index("s")

    @pl.when(cid == 0)
    def _():
        base = sid * STRIPE
        pltpu.sync_copy(cnt_hbm.at[pl.ds(base, STRIPE)], n0)
        pltpu.sync_copy(cnt_hbm.at[pl.ds(CLS_PAD + base, STRIPE)], n1)
        pltpu.sync_copy(dsum_hbm.at[pl.ds(base, STRIPE)], d0)
        pltpu.sync_copy(dsum_hbm.at[pl.ds(CLS_PAD + base, STRIPE)], d1)

        def rstep(i, acc):
            n = n0[pl.ds(i * 16, 16)] + n1[pl.ds(i * 16, 16)]
            dv = d0[pl.ds(i * 16, 16)] + d1[pl.ds(i * 16, 16)]
            return acc + dv / (n + n + 2.0)
        accv = lax.fori_loop(0, STRIPE // 16, rstep,
                             jnp.zeros((16,), jnp.float32))
        fin_v[0, pl.ds(0, 16)] = accv
        pltpu.sync_copy(fin_v.at[0], part_sh.at[sid])

        plsc.subcore_barrier()

        @pl.when(sid == 0)
        def _():
            pltpu.sync_copy(part_sh, fin_v)
            tot = jnp.zeros((16,), jnp.float32)
            for r in range(NSUB):
                tot = tot + fin_v[r, pl.ds(0, 16)]
            tot = plsc.cumsum(tot) * (1.0 / B)
            fin_v[0, pl.ds(0, 16)] = tot
            pltpu.sync_copy(fin_v.at[0], out_hbm)


@jax.jit
def _center_loss(xs, ys, center):
    main_k = pl.kernel(
        _main_body,
        out_type=(
            jax.ShapeDtypeStruct((NCORE * CLS_PAD,), jnp.float32),
            jax.ShapeDtypeStruct((NCORE * CLS_PAD,), jnp.float32),
        ),
        mesh=plsc.VectorSubcoreMesh(core_axis_name="c", subcore_axis_name="s",
                                    num_cores=NCORE),
        compiler_params=pltpu.CompilerParams(
            needs_layout_passes=False, use_tc_tiling_on_sc=True),
        scratch_types=[
            pltpu.VMEM_SHARED((CLS_PAD,), jnp.float32),
            pltpu.VMEM_SHARED((CLS_PAD,), jnp.float32),
            pltpu.VMEM((NBATCH, BSZ), jnp.int32),
            pltpu.VMEM((2 * BSZ * FEAT,), jnp.float32),
            pltpu.VMEM((2 * BSZ, 8, FEAT), jnp.float32),
            pltpu.VMEM((1, BSZ), jnp.float32),
            pltpu.VMEM((1, BSZ), jnp.float32),
            pltpu.VMEM((STRIPE,), jnp.float32),
            pltpu.SemaphoreType.DMA,
            pltpu.SemaphoreType.DMA,
        ],
    )
    red_k = pl.kernel(
        _reduce_body,
        out_type=jax.ShapeDtypeStruct((16,), jnp.float32),
        mesh=plsc.VectorSubcoreMesh(core_axis_name="c", subcore_axis_name="s",
                                    num_cores=NCORE),
        compiler_params=pltpu.CompilerParams(
            needs_layout_passes=False, use_tc_tiling_on_sc=False),
        scratch_types=[
            pltpu.VMEM_SHARED((NSUB, 16), jnp.float32),
            pltpu.VMEM((STRIPE,), jnp.float32),
            pltpu.VMEM((STRIPE,), jnp.float32),
            pltpu.VMEM((STRIPE,), jnp.float32),
            pltpu.VMEM((STRIPE,), jnp.float32),
            pltpu.VMEM((NSUB, 16), jnp.float32),
        ],
    )
    cnt, dsum = main_k(xs.reshape(-1), ys, center)
    return red_k(cnt, dsum)


def kernel(xs, ys, center):
    out = _center_loss(xs, ys.astype(jnp.int32), center)
    return out[15]


# speedup vs baseline: 1.0366x; 1.0366x over previous
import functools

import jax
import jax.numpy as jnp
from jax import lax
from jax.experimental import pallas as pl
from jax.experimental.pallas import tpu as pltpu
from jax.experimental.pallas import tpu_sc as plsc

CLS = 100000
FEAT = 64
B = 16384
NBLK = CLS // 8

NCORE = 2
NSUB = 16
NW = NCORE * NSUB
CHUNK = B // NW
BSZ = 128
NBATCH = CHUNK // BSZ
CLS_PAD = 100352
STRIPE = CLS_PAD // NSUB


def _main_body(xs_hbm, ys_hbm, center_hbm, cnt_hbm, dsum_hbm,
               cnt_sh, dsum_sh,
               idx_v, xs_v, tiles, dvals, ones_v, stage, sem, semx):
    cid = lax.axis_index("c")
    sid = lax.axis_index("s")
    wid = cid * NSUB + sid
    zero16 = jnp.zeros((16,), jnp.float32)
    one16 = jnp.ones((16,), jnp.float32)
    lane = lax.iota(jnp.int32, 16)

    def fire(b):
        pltpu.async_copy(
            xs_hbm.at[pl.ds((wid * CHUNK + b * BSZ) * FEAT, BSZ * FEAT)],
            xs_v.at[pl.ds((b & 1) * BSZ * FEAT, BSZ * FEAT)], semx)
        pltpu.async_copy(
            center_hbm.at[idx_v.at[b]],
            tiles.at[pl.ds((b & 1) * BSZ, BSZ)], sem)

    def drain(b):
        pltpu.make_async_copy(
            xs_hbm.at[pl.ds(0, BSZ * FEAT)],
            xs_v.at[pl.ds((b & 1) * BSZ * FEAT, BSZ * FEAT)], semx).wait()
        pltpu.make_async_copy(
            center_hbm.at[pl.ds(0, BSZ)],
            tiles.at[pl.ds((b & 1) * BSZ, BSZ)], sem).wait()

    for b in range(NBATCH):
        pltpu.sync_copy(ys_hbm.at[pl.ds(wid * CHUNK + b * BSZ, BSZ)],
                        idx_v.at[b])

    def zfill(i, _):
        stage[pl.ds(i * 16, 16)] = zero16
        return 0
    lax.fori_loop(0, STRIPE // 16, zfill, 0)
    pltpu.sync_copy(stage.at[pl.ds(0, STRIPE)],
                    cnt_sh.at[pl.ds(sid * STRIPE, STRIPE)])
    pltpu.sync_copy(stage.at[pl.ds(0, STRIPE)],
                    dsum_sh.at[pl.ds(sid * STRIPE, STRIPE)])
    for k in range(BSZ // 16):
        ones_v[0, pl.ds(k * 16, 16)] = one16

    fire(0)
    plsc.subcore_barrier()

    def batch_step(b, _):
        @pl.when(b + 1 < NBATCH)
        def _():
            fire(b + 1)
        drain(b)
        ring = (b & 1) * BSZ

        def dgroup(k, _):
            slot = ring + k * 16 + lane
            xbase = ((b & 1) * BSZ + k * 16 + lane) * FEAT
            acc = jnp.zeros((16,), jnp.float32)
            for f0 in range(FEAT):
                fd = (lane + f0) & (FEAT - 1)
                xv = plsc.load_gather(xs_v, [xbase + fd])
                cv = plsc.load_gather(tiles, [slot, fd])
                df = xv - cv
                acc = acc + df * df
            dvals[0, pl.ds(k * 16, 16)] = acc
            return 0
        lax.fori_loop(0, BSZ // 16, dgroup, 0)

        pltpu.sync_copy(ones_v.at[0], cnt_sh.at[idx_v.at[b]], add=True)
        pltpu.sync_copy(dvals.at[0], dsum_sh.at[idx_v.at[b]], add=True)
        return 0
    lax.fori_loop(0, NBATCH, batch_step, 0)

    plsc.subcore_barrier()

    pltpu.sync_copy(cnt_sh.at[pl.ds(sid * STRIPE, STRIPE)], stage)
    pltpu.sync_copy(stage,
                    cnt_hbm.at[pl.ds(cid * CLS_PAD + sid * STRIPE, STRIPE)])
    pltpu.sync_copy(dsum_sh.at[pl.ds(sid * STRIPE, STRIPE)], stage)
    pltpu.sync_copy(stage,
                    dsum_hbm.at[pl.ds(cid * CLS_PAD + sid * STRIPE, STRIPE)])


def _reduce_body(cnt_hbm, dsum_hbm, out_hbm,
                 part_sh, n0, n1, d0, d1, fin_v):
    cid = lax.axis_index("c")
    sid = lax.axis_index("s")

    @pl.when(cid == 0)
    def _():
        base = sid * STRIPE
        pltpu.sync_copy(cnt_hbm.at[pl.ds(base, STRIPE)], n0)
        pltpu.sync_copy(cnt_hbm.at[pl.ds(CLS_PAD + base, STRIPE)], n1)
        pltpu.sync_copy(dsum_hbm.at[pl.ds(base, STRIPE)], d0)
        pltpu.sync_copy(dsum_hbm.at[pl.ds(CLS_PAD + base, STRIPE)], d1)

        def rstep(i, acc):
            n = n0[pl.ds(i * 16, 16)] + n1[pl.ds(i * 16, 16)]
            dv = d0[pl.ds(i * 16, 16)] + d1[pl.ds(i * 16, 16)]
            return acc + dv / (n + n + 2.0)
        accv = lax.fori_loop(0, STRIPE // 16, rstep,
                             jnp.zeros((16,), jnp.float32))
        fin_v[0, pl.ds(0, 16)] = accv
        pltpu.sync_copy(fin_v.at[0], part_sh.at[sid])

        plsc.subcore_barrier()

        @pl.when(sid == 0)
        def _():
            pltpu.sync_copy(part_sh, fin_v)
            tot = jnp.zeros((16,), jnp.float32)
            for r in range(NSUB):
                tot = tot + fin_v[r, pl.ds(0, 16)]
            tot = plsc.cumsum(tot) * (1.0 / B)
            fin_v[0, pl.ds(0, 16)] = tot
            pltpu.sync_copy(fin_v.at[0], out_hbm)


@jax.jit
def _center_loss(xs, ys, center):
    main_k = pl.kernel(
        _main_body,
        out_type=(
            jax.ShapeDtypeStruct((NCORE * CLS_PAD,), jnp.float32),
            jax.ShapeDtypeStruct((NCORE * CLS_PAD,), jnp.float32),
        ),
        mesh=plsc.VectorSubcoreMesh(core_axis_name="c", subcore_axis_name="s",
                                    num_cores=NCORE),
        compiler_params=pltpu.CompilerParams(
            needs_layout_passes=False, use_tc_tiling_on_sc=False),
        scratch_types=[
            pltpu.VMEM_SHARED((CLS_PAD,), jnp.float32),
            pltpu.VMEM_SHARED((CLS_PAD,), jnp.float32),
            pltpu.VMEM((NBATCH, BSZ), jnp.int32),
            pltpu.VMEM((2 * BSZ * FEAT,), jnp.float32),
            pltpu.VMEM((2 * BSZ, FEAT), jnp.float32),
            pltpu.VMEM((1, BSZ), jnp.float32),
            pltpu.VMEM((1, BSZ), jnp.float32),
            pltpu.VMEM((STRIPE,), jnp.float32),
            pltpu.SemaphoreType.DMA,
            pltpu.SemaphoreType.DMA,
        ],
    )
    red_k = pl.kernel(
        _reduce_body,
        out_type=jax.ShapeDtypeStruct((16,), jnp.float32),
        mesh=plsc.VectorSubcoreMesh(core_axis_name="c", subcore_axis_name="s",
                                    num_cores=NCORE),
        compiler_params=pltpu.CompilerParams(
            needs_layout_passes=False, use_tc_tiling_on_sc=False),
        scratch_types=[
            pltpu.VMEM_SHARED((NSUB, 16), jnp.float32),
            pltpu.VMEM((STRIPE,), jnp.float32),
            pltpu.VMEM((STRIPE,), jnp.float32),
            pltpu.VMEM((STRIPE,), jnp.float32),
            pltpu.VMEM((STRIPE,), jnp.float32),
            pltpu.VMEM((NSUB, 16), jnp.float32),
        ],
    )
    cnt, dsum = main_k(xs.reshape(-1), ys, center)
    return red_k(cnt, dsum)


def kernel(xs, ys, center):
    out = _center_loss(xs, ys.astype(jnp.int32), center)
    return out[15]


# speedup vs baseline: 1.1874x; 1.1454x over previous
import functools

import jax
import jax.numpy as jnp
from jax import lax
from jax.experimental import pallas as pl
from jax.experimental.pallas import tpu as pltpu
from jax.experimental.pallas import tpu_sc as plsc

CLS = 100000
FEAT = 64
B = 16384
NBLK = CLS // 8

NCORE = 2
NSUB = 16
NW = NCORE * NSUB
CHUNK = B // NW
BSZ = 32
NBATCH = CHUNK // BSZ
CLS_PAD = 100352
STRIPE = CLS_PAD // NSUB


def _main_body(xs_hbm, ys_hbm, center_hbm, cnt_hbm, dsum_hbm,
               cnt_sh, dsum_sh,
               idx_v, xs_v, tiles, dvals, ones_v, stage, sem, semx):
    cid = lax.axis_index("c")
    sid = lax.axis_index("s")
    wid = cid * NSUB + sid
    zero16 = jnp.zeros((16,), jnp.float32)
    one16 = jnp.ones((16,), jnp.float32)
    lane = lax.iota(jnp.int32, 16)

    def fire(b):
        pltpu.async_copy(
            xs_hbm.at[pl.ds((wid * CHUNK + b * BSZ) * FEAT, BSZ * FEAT)],
            xs_v.at[pl.ds((b & 1) * BSZ * FEAT, BSZ * FEAT)], semx)
        ring = (b & 1) * BSZ
        for k in range(BSZ // 16):
            yv = idx_v[b, pl.ds(k * 16, 16)]
            for l in range(16):
                y = yv[l]
                pltpu.async_copy(center_hbm.at[y >> 3],
                                 tiles.at[ring + k * 16 + l], sem)

    def drain(b):
        pltpu.make_async_copy(
            xs_hbm.at[pl.ds(0, BSZ * FEAT)],
            xs_v.at[pl.ds((b & 1) * BSZ * FEAT, BSZ * FEAT)], semx).wait()
        for n in range(BSZ):
            pltpu.make_async_copy(
                center_hbm.at[0],
                tiles.at[(b & 1) * BSZ + n], sem).wait()

    for b in range(NBATCH):
        pltpu.sync_copy(ys_hbm.at[pl.ds(wid * CHUNK + b * BSZ, BSZ)],
                        idx_v.at[b])

    def zfill(i, _):
        stage[pl.ds(i * 16, 16)] = zero16
        return 0
    lax.fori_loop(0, STRIPE // 16, zfill, 0)
    pltpu.sync_copy(stage.at[pl.ds(0, STRIPE)],
                    cnt_sh.at[pl.ds(sid * STRIPE, STRIPE)])
    pltpu.sync_copy(stage.at[pl.ds(0, STRIPE)],
                    dsum_sh.at[pl.ds(sid * STRIPE, STRIPE)])
    for k in range(BSZ // 16):
        ones_v[0, pl.ds(k * 16, 16)] = one16

    fire(0)
    plsc.subcore_barrier()

    def batch_step(b, _):
        @pl.when(b + 1 < NBATCH)
        def _():
            fire(b + 1)
        drain(b)
        ring = (b & 1) * BSZ

        def dgroup(k, _):
            yv = idx_v[b, pl.ds(k * 16, 16)]
            slot = ring + k * 16 + lane
            row = yv & 7
            xbase = ((b & 1) * BSZ + k * 16 + lane) * FEAT
            acc = jnp.zeros((16,), jnp.float32)
            for f0 in range(FEAT):
                fd = (lane + f0) & (FEAT - 1)
                xv = plsc.load_gather(xs_v, [xbase + fd])
                cv = plsc.load_gather(tiles, [slot, row, fd])
                df = xv - cv
                acc = acc + df * df
            dvals[0, pl.ds(k * 16, 16)] = acc
            return 0
        lax.fori_loop(0, BSZ // 16, dgroup, 0)

        pltpu.sync_copy(ones_v.at[0], cnt_sh.at[idx_v.at[b]], add=True)
        pltpu.sync_copy(dvals.at[0], dsum_sh.at[idx_v.at[b]], add=True)
        return 0
    lax.fori_loop(0, NBATCH, batch_step, 0)

    plsc.subcore_barrier()

    pltpu.sync_copy(cnt_sh.at[pl.ds(sid * STRIPE, STRIPE)], stage)
    pltpu.sync_copy(stage,
                    cnt_hbm.at[pl.ds(cid * CLS_PAD + sid * STRIPE, STRIPE)])
    pltpu.sync_copy(dsum_sh.at[pl.ds(sid * STRIPE, STRIPE)], stage)
    pltpu.sync_copy(stage,
                    dsum_hbm.at[pl.ds(cid * CLS_PAD + sid * STRIPE, STRIPE)])


def _reduce_body(cnt_hbm, dsum_hbm, out_hbm,
                 part_sh, n0, n1, d0, d1, fin_v):
    cid = lax.axis_index("c")
    sid = lax.axis_index("s")

    @pl.when(cid == 0)
    def _():
        base = sid * STRIPE
        pltpu.sync_copy(cnt_hbm.at[pl.ds(base, STRIPE)], n0)
        pltpu.sync_copy(cnt_hbm.at[pl.ds(CLS_PAD + base, STRIPE)], n1)
        pltpu.sync_copy(dsum_hbm.at[pl.ds(base, STRIPE)], d0)
        pltpu.sync_copy(dsum_hbm.at[pl.ds(CLS_PAD + base, STRIPE)], d1)

        def rstep(i, acc):
            n = n0[pl.ds(i * 16, 16)] + n1[pl.ds(i * 16, 16)]
            dv = d0[pl.ds(i * 16, 16)] + d1[pl.ds(i * 16, 16)]
            return acc + dv / (n + n + 2.0)
        accv = lax.fori_loop(0, STRIPE // 16, rstep,
                             jnp.zeros((16,), jnp.float32))
        fin_v[0, pl.ds(0, 16)] = accv
        pltpu.sync_copy(fin_v.at[0], part_sh.at[sid])

        plsc.subcore_barrier()

        @pl.when(sid == 0)
        def _():
            pltpu.sync_copy(part_sh, fin_v)
            tot = jnp.zeros((16,), jnp.float32)
            for r in range(NSUB):
                tot = tot + fin_v[r, pl.ds(0, 16)]
            tot = plsc.cumsum(tot) * (1.0 / B)
            fin_v[0, pl.ds(0, 16)] = tot
            pltpu.sync_copy(fin_v.at[0], out_hbm)


@jax.jit
def _center_loss(xs, ys, center):
    main_k = pl.kernel(
        _main_body,
        out_type=(
            jax.ShapeDtypeStruct((NCORE * CLS_PAD,), jnp.float32),
            jax.ShapeDtypeStruct((NCORE * CLS_PAD,), jnp.float32),
        ),
        mesh=plsc.VectorSubcoreMesh(core_axis_name="c", subcore_axis_name="s",
                                    num_cores=NCORE),
        compiler_params=pltpu.CompilerParams(
            needs_layout_passes=False, use_tc_tiling_on_sc=True),
        scratch_types=[
            pltpu.VMEM_SHARED((CLS_PAD,), jnp.float32),
            pltpu.VMEM_SHARED((CLS_PAD,), jnp.float32),
            pltpu.VMEM((NBATCH, BSZ), jnp.int32),
            pltpu.VMEM((2 * BSZ * FEAT,), jnp.float32),
            pltpu.VMEM((2 * BSZ, 8, FEAT), jnp.float32),
            pltpu.VMEM((1, BSZ), jnp.float32),
            pltpu.VMEM((1, BSZ), jnp.float32),
            pltpu.VMEM((STRIPE,), jnp.float32),
            pltpu.SemaphoreType.DMA,
            pltpu.SemaphoreType.DMA,
        ],
    )
    red_k = pl.kernel(
        _reduce_body,
        out_type=jax.ShapeDtypeStruct((16,), jnp.float32),
        mesh=plsc.VectorSubcoreMesh(core_axis_name="c", subcore_axis_name="s",
                                    num_cores=NCORE),
        compiler_params=pltpu.CompilerParams(
            needs_layout_passes=False, use_tc_tiling_on_sc=False),
        scratch_types=[
            pltpu.VMEM_SHARED((NSUB, 16), jnp.float32),
            pltpu.VMEM((STRIPE,), jnp.float32),
            pltpu.VMEM((STRIPE,), jnp.float32),
            pltpu.VMEM((STRIPE,), jnp.float32),
            pltpu.VMEM((STRIPE,), jnp.float32),
            pltpu.VMEM((NSUB, 16), jnp.float32),
        ],
    )
    center3 = center.reshape(NBLK, 8, FEAT)
    cnt, dsum = main_k(xs.reshape(-1), ys, center3)
    return red_k(cnt, dsum)


def kernel(xs, ys, center):
    out = _center_loss(xs, ys.astype(jnp.int32), center)
    return out[15]
